# baseline (speedup 1.0000x reference)
import functools
import math

import jax
import jax.numpy as jnp
from jax import lax
from jax.experimental import pallas as pl
from jax.experimental.pallas import tpu as pltpu
from jax.experimental.pallas import tpu_sc as plsc

N = 10000
E = 160000
D = 256
HD = 128
ACC_W = 144
NB = 10
RB = N // NB

NUM_TILES = 16
NP = N + 8
CHUNK = 48
EPT = 10080
EP = EPT * NUM_TILES
NCH = EPT // CHUNK
ROWS_PER_TILE = N // NUM_TILES


def _proj_body(h_ref, wq, bq, wk, bk, wv, bv, wm, bm, wat, bat,
               q0, q1, k0, k1, v0, v1):
    hb = h_ref[:, :]
    f32 = jnp.float32
    q = jnp.dot(hb, wq[:, :], preferred_element_type=f32) + bq[:, :]
    kt = jnp.dot(hb, wk[:, :], preferred_element_type=f32) + bk[:, :]
    k = jnp.dot(kt, wat[:, :], preferred_element_type=f32) + bat[:, :]
    vt = jnp.dot(hb, wv[:, :], preferred_element_type=f32) + bv[:, :]
    v = jnp.dot(vt, wm[:, :], preferred_element_type=f32) + bm[:, :]
    q0[:, :] = q[:, :HD]
    q1[:, :] = q[:, HD:]
    k0[:, :] = k[:, :HD]
    k1[:, :] = k[:, HD:]
    v0[:, :] = v[:, :HD]
    v1[:, :] = v[:, HD:]


def _project(h, Wq, bq, Wk, bk, Wv, bv, Wm, bm, Wat, bat):
    wspec = pl.BlockSpec((D, D), lambda i: (0, 0))
    bspec = pl.BlockSpec((1, D), lambda i: (0, 0))
    hspec = pl.BlockSpec((RB, D), lambda i: (i, 0))
    ospec = pl.BlockSpec((RB, HD), lambda i: (i, 0))
    out = jax.ShapeDtypeStruct((N, HD), jnp.float32)
    return pl.pallas_call(
        _proj_body,
        grid=(NB,),
        in_specs=[hspec, wspec, bspec, wspec, bspec, wspec, bspec,
                  wspec, bspec, wspec, bspec],
        out_specs=[ospec] * 6,
        out_shape=[out] * 6,
    )(h, Wq, bq.reshape(1, D), Wk, bk.reshape(1, D), Wv, bv.reshape(1, D),
      Wm, bm.reshape(1, D), Wat, bat.reshape(1, D))


def _sc_half(k_hbm, q_hbm, v_hbm, src_hbm, dst_hbm, outv_hbm, outd_hbm,
             accv, accd, src_i, dst_i, dst_s, kb, qb, vb, exb, gsem, vsem, ssem):
    sid = lax.axis_index("s")
    zeros16 = jnp.zeros((16,), jnp.float32)
    lanes = jnp.arange(16, dtype=jnp.int32)

    def zrow(e, _):
        for j in range(HD // 16):
            vb[0][e, pl.ds(j * 16, 16)] = zeros16
        exb[0][e, :] = zeros16
        return _
    lax.fori_loop(0, CHUNK, zrow, None)

    row0 = sid * ROWS_PER_TILE
    def zacc(j, _):
        pltpu.sync_copy(vb[0].at[pl.ds(0, CHUNK)],
                        accv.at[pl.ds(row0 + j * CHUNK, CHUNK)])
        pltpu.sync_copy(exb[0].at[pl.ds(0, CHUNK)],
                        accd.at[pl.ds(row0 + j * CHUNK, CHUNK)])
        return _
    lax.fori_loop(0, 13, zacc, None)
    pltpu.sync_copy(vb[0].at[pl.ds(0, 1)], accv.at[pl.ds(row0 + 624, 1)])
    pltpu.sync_copy(exb[0].at[pl.ds(0, 1)], accd.at[pl.ds(row0 + 624, 1)])

    ebase = sid * EPT

    pltpu.sync_copy(src_hbm.at[pl.ds(ebase, CHUNK)], src_i[0])
    pltpu.sync_copy(dst_hbm.at[pl.ds(ebase, CHUNK)], dst_i[0])
    pltpu.async_copy(k_hbm.at[src_i[0]], kb[0], gsem[0])
    pltpu.async_copy(q_hbm.at[dst_i[0]], qb[0], gsem[0])
    pltpu.async_copy(v_hbm.at[src_i[0]], vb[0], vsem[0])
    plsc.subcore_barrier()

    def pair_body(p, _):
        for b in (0, 1):
            o = 1 - b
            g = 2 * p + b

            @pl.when(g < NCH - 1)
            def _():
                nbase = ebase + (g + 1) * CHUNK
                pltpu.sync_copy(src_hbm.at[pl.ds(nbase, CHUNK)], src_i[o])
                pltpu.sync_copy(dst_hbm.at[pl.ds(nbase, CHUNK)], dst_i[o])
                pltpu.async_copy(k_hbm.at[src_i[o]], kb[o], gsem[o])
                pltpu.async_copy(q_hbm.at[dst_i[o]], qb[o], gsem[o])

            pltpu.make_async_copy(k_hbm.at[src_i[b]], kb[b], gsem[b]).wait()
            pltpu.make_async_copy(q_hbm.at[dst_i[b]], qb[b], gsem[b]).wait()

            for gg in range(CHUNK // 16):
                eidx = gg * 16 + lanes

                def d_body(d, accs):
                    new_accs = []
                    for h in range(8):
                        cols = jnp.full((16,), h * 16, jnp.int32) + d
                        kv = plsc.load_gather(kb[b], [eidx, cols])
                        qv = plsc.load_gather(qb[b], [eidx, cols])
                        new_accs.append(accs[h] + kv * qv)
                    return tuple(new_accs)
                accs = lax.fori_loop(0, 16, d_body, (zeros16,) * 8)
                for h in range(8):
                    ex = jnp.exp(accs[h] * 0.25)
                    plsc.store_scatter(
                        exb[b], [eidx, jnp.full((16,), h, jnp.int32)], ex)

            @pl.when(g >= 1)
            def _():
                pltpu.make_async_copy(vb[o], accv.at[dst_s[o]], ssem[o]).wait()
                pltpu.make_async_copy(exb[o], accd.at[dst_s[o]], ssem[o]).wait()

            @pl.when(g < NCH - 1)
            def _():
                pltpu.async_copy(v_hbm.at[src_i[o]], vb[o], vsem[o])

            pltpu.make_async_copy(v_hbm.at[src_i[b]], vb[b], vsem[b]).wait()

            def wt_body(e, _):
                ev = exb[b][e, :]
                for h in range(8):
                    x = ev[h]
                    vb[b][e, pl.ds(h * 16, 16)] = vb[b][e, pl.ds(h * 16, 16)] * x
                return _
            lax.fori_loop(0, CHUNK, wt_body, None)

            for j in range(CHUNK // 16):
                dst_s[b][pl.ds(j * 16, 16)] = dst_i[b][pl.ds(j * 16, 16)]

            pltpu.async_copy(vb[b], accv.at[dst_s[b]], ssem[b], add=True)
            pltpu.async_copy(exb[b], accd.at[dst_s[b]], ssem[b], add=True)
        return _

    lax.fori_loop(0, NCH // 2, pair_body, None)

    pltpu.make_async_copy(vb[1], accv.at[dst_s[1]], ssem[1]).wait()
    pltpu.make_async_copy(exb[1], accd.at[dst_s[1]], ssem[1]).wait()
    plsc.subcore_barrier()

    pltpu.sync_copy(accv.at[pl.ds(row0, ROWS_PER_TILE)],
                    outv_hbm.at[pl.ds(row0, ROWS_PER_TILE)])
    pltpu.sync_copy(accd.at[pl.ds(row0, ROWS_PER_TILE)],
                    outd_hbm.at[pl.ds(row0, ROWS_PER_TILE)])


def _sc_body(k0, q0, v0, k1, q1, v1, src_hbm, dst_hbm,
             numv0, numd0, numv1, numd1,
             accv, accd,
             src_i0, src_i1, dst_i0, dst_i1, dst_s0, dst_s1,
             kb0, kb1, qb0, qb1, vb0, vb1, exb0, exb1,
             gsem0, gsem1, vsem0, vsem1, ssem0, ssem1):
    cid = lax.axis_index("c")
    args = (accv, accd, (src_i0, src_i1), (dst_i0, dst_i1), (dst_s0, dst_s1),
            (kb0, kb1), (qb0, qb1), (vb0, vb1), (exb0, exb1),
            (gsem0, gsem1), (vsem0, vsem1), (ssem0, ssem1))

    @pl.when(cid == 0)
    def _():
        _sc_half(k0, q0, v0, src_hbm, dst_hbm, numv0, numd0, *args)

    @pl.when(cid == 1)
    def _():
        _sc_half(k1, q1, v1, src_hbm, dst_hbm, numv1, numd1, *args)


def _sc_edges(k0, q0, v0, k1, q1, v1, src, dst):
    mesh = plsc.VectorSubcoreMesh(core_axis_name="c", subcore_axis_name="s")
    outv = jax.ShapeDtypeStruct((N, HD), jnp.float32)
    outd = jax.ShapeDtypeStruct((N, 16), jnp.float32)
    idx_t = pltpu.VMEM((CHUNK,), jnp.int32)
    row_t = pltpu.VMEM((CHUNK, HD), jnp.float32)
    ex_t = pltpu.VMEM((CHUNK, 16), jnp.float32)
    sem_t = pltpu.SemaphoreType.DMA
    fn = pl.kernel(
        _sc_body,
        mesh=mesh,
        out_type=[outv, outd, outv, outd],
        compiler_params=pltpu.CompilerParams(use_tc_tiling_on_sc=False,
                                             needs_layout_passes=False,
                                             disable_bounds_checks=True),
        scratch_types=[
            pltpu.VMEM_SHARED((NP, HD), jnp.float32),
            pltpu.VMEM_SHARED((NP, 16), jnp.float32),
            idx_t, idx_t, idx_t, idx_t, idx_t, idx_t,
            row_t, row_t, row_t, row_t, row_t, row_t, ex_t, ex_t,
            sem_t, sem_t, sem_t, sem_t, sem_t, sem_t,
        ],
    )
    return fn(k0, q0, v0, k1, q1, v1, src, dst)


def _out_body(nv0_ref, nd0_ref, nv1_ref, nd1_ref, wa, ba, out_ref):
    f32 = jnp.float32
    row = lax.broadcasted_iota(jnp.int32, (8, HD), 0)
    col = lax.broadcasted_iota(jnp.int32, (8, HD), 1)
    expand = (col // 16 == row).astype(f32)
    r0 = 1.0 / jnp.maximum(nd0_ref[:, :8], 1e-30)
    r1 = 1.0 / jnp.maximum(nd1_ref[:, :8], 1e-30)
    att0 = nv0_ref[:, :] * jnp.dot(r0, expand, preferred_element_type=f32)
    att1 = nv1_ref[:, :] * jnp.dot(r1, expand, preferred_element_type=f32)
    out = (jnp.dot(att0, wa[:HD, :], preferred_element_type=f32)
           + jnp.dot(att1, wa[HD:, :], preferred_element_type=f32)
           + ba[:, :])
    out_ref[:, :] = out


def _output(nv0, nd0, nv1, nd1, Wa, ba):
    vspec = pl.BlockSpec((RB, HD), lambda i: (i, 0))
    dspec = pl.BlockSpec((RB, 16), lambda i: (i, 0))
    return pl.pallas_call(
        _out_body,
        grid=(NB,),
        in_specs=[vspec, dspec, vspec, dspec,
                  pl.BlockSpec((D, D), lambda i: (0, 0)),
                  pl.BlockSpec((1, D), lambda i: (0, 0))],
        out_specs=pl.BlockSpec((RB, D), lambda i: (i, 0)),
        out_shape=jax.ShapeDtypeStruct((N, D), jnp.float32),
    )(nv0, nd0, nv1, nd1, Wa, ba.reshape(1, D))


def kernel(h, edge_index, Wq, bq, Wk, bk, Wv, bv, Wm, bm, Wat, bat, Wa, ba):
    pad = jnp.full((EP - E,), N, jnp.int32)
    src = jnp.concatenate([edge_index[0], pad])
    dst = jnp.concatenate([edge_index[1], pad])
    q0, q1, k0, k1, v0, v1 = _project(h, Wq, bq, Wk, bk, Wv, bv,
                                      Wm, bm, Wat, bat)
    zrows = jnp.zeros((NP - N, HD), jnp.float32)
    q0, q1, k0, k1, v0, v1 = (jnp.concatenate([a, zrows])
                              for a in (q0, q1, k0, k1, v0, v1))
    nv0, nd0, nv1, nd1 = _sc_edges(k0, q0, v0, k1, q1, v1, src, dst)
    return _output(nv0, nd0, nv1, nd1, Wa, ba)


# speedup vs baseline: 1.4583x; 1.4583x over previous
import functools
import math

import jax
import jax.numpy as jnp
from jax import lax
from jax.experimental import pallas as pl
from jax.experimental.pallas import tpu as pltpu
from jax.experimental.pallas import tpu_sc as plsc

N = 10000
E = 160000
D = 256
HD = 128
ACC_W = 144
NB = 10
RB = N // NB

NUM_TILES = 16
NP = N + 8
CHUNK = 32
EPT = 10048
EP = EPT * NUM_TILES
NCH = EPT // CHUNK
ROWS_PER_TILE = N // NUM_TILES


def _proj_body(h_ref, wq, bq, wk, bk, wv, bv, wm, bm, wat, bat,
               q0, q1, k0, k1, v0, v1):
    hb = h_ref[:, :]
    f32 = jnp.float32
    q = jnp.dot(hb, wq[:, :], preferred_element_type=f32) + bq[:, :]
    kt = jnp.dot(hb, wk[:, :], preferred_element_type=f32) + bk[:, :]
    k = jnp.dot(kt, wat[:, :], preferred_element_type=f32) + bat[:, :]
    vt = jnp.dot(hb, wv[:, :], preferred_element_type=f32) + bv[:, :]
    v = jnp.dot(vt, wm[:, :], preferred_element_type=f32) + bm[:, :]
    q0[:, :] = q[:, :HD]
    q1[:, :] = q[:, HD:]
    k0[:, :] = k[:, :HD]
    k1[:, :] = k[:, HD:]
    v0[:, :] = v[:, :HD]
    v1[:, :] = v[:, HD:]


def _project(h, Wq, bq, Wk, bk, Wv, bv, Wm, bm, Wat, bat):
    wspec = pl.BlockSpec((D, D), lambda i: (0, 0))
    bspec = pl.BlockSpec((1, D), lambda i: (0, 0))
    hspec = pl.BlockSpec((RB, D), lambda i: (i, 0))
    ospec = pl.BlockSpec((RB, HD), lambda i: (i, 0))
    out = jax.ShapeDtypeStruct((N, HD), jnp.float32)
    return pl.pallas_call(
        _proj_body,
        grid=(NB,),
        in_specs=[hspec, wspec, bspec, wspec, bspec, wspec, bspec,
                  wspec, bspec, wspec, bspec],
        out_specs=[ospec] * 6,
        out_shape=[out] * 6,
    )(h, Wq, bq.reshape(1, D), Wk, bk.reshape(1, D), Wv, bv.reshape(1, D),
      Wm, bm.reshape(1, D), Wat, bat.reshape(1, D))


def _sc_half(k_hbm, q_hbm, v_hbm, src_hbm, dst_hbm, outv_hbm, outd_hbm,
             accv, accd, src_i, dst_i, dst_s, kb, qb, vb, exb, kp, qp,
             gsem, vsem, ssem):
    sid = lax.axis_index("s")
    zeros16 = jnp.zeros((16,), jnp.float32)
    lanes = jnp.arange(16, dtype=jnp.int32)

    def zrow(e, _):
        for j in range(HD // 16):
            vb[0][e, pl.ds(j * 16, 16)] = zeros16
        exb[0][e, :] = zeros16
        return _
    lax.fori_loop(0, CHUNK, zrow, None)

    row0 = sid * ROWS_PER_TILE
    def zacc(j, _):
        pltpu.sync_copy(vb[0].at[pl.ds(0, CHUNK)],
                        accv.at[pl.ds(row0 + j * CHUNK, CHUNK)])
        pltpu.sync_copy(exb[0].at[pl.ds(0, CHUNK)],
                        accd.at[pl.ds(row0 + j * CHUNK, CHUNK)])
        return _
    lax.fori_loop(0, 19, zacc, None)
    pltpu.sync_copy(vb[0].at[pl.ds(0, 17)], accv.at[pl.ds(row0 + 608, 17)])
    pltpu.sync_copy(exb[0].at[pl.ds(0, 17)], accd.at[pl.ds(row0 + 608, 17)])

    ebase = sid * EPT

    pltpu.sync_copy(src_hbm.at[pl.ds(ebase, CHUNK)], src_i[0])
    pltpu.sync_copy(dst_hbm.at[pl.ds(ebase, CHUNK)], dst_i[0])
    pltpu.async_copy(k_hbm.at[src_i[0]], kb[0], gsem[0])
    pltpu.async_copy(q_hbm.at[dst_i[0]], qb[0], gsem[0])
    pltpu.async_copy(v_hbm.at[src_i[0]], vb[0], vsem[0])
    plsc.subcore_barrier()

    def pair_body(p, _):
        for b in (0, 1):
            o = 1 - b
            g = 2 * p + b

            @pl.when(g < NCH - 1)
            def _():
                nbase = ebase + (g + 1) * CHUNK
                pltpu.sync_copy(src_hbm.at[pl.ds(nbase, CHUNK)], src_i[o])
                pltpu.sync_copy(dst_hbm.at[pl.ds(nbase, CHUNK)], dst_i[o])
                pltpu.async_copy(k_hbm.at[src_i[o]], kb[o], gsem[o])
                pltpu.async_copy(q_hbm.at[dst_i[o]], qb[o], gsem[o])

            pltpu.make_async_copy(k_hbm.at[src_i[b]], kb[b], gsem[b]).wait()
            pltpu.make_async_copy(q_hbm.at[dst_i[b]], qb[b], gsem[b]).wait()
            def rs_body(e, _):
                for j in range(8):
                    kp[pl.ds(e * 129 + j * 16, 16)] = kb[b][e, pl.ds(j * 16, 16)]
                    qp[pl.ds(e * 129 + j * 16, 16)] = qb[b][e, pl.ds(j * 16, 16)]
                return _
            lax.fori_loop(0, CHUNK, rs_body, None)

            for gg in range(CHUNK // 16):
                eidx = gg * 16 + lanes

                eidx129 = eidx * 129

                def d_body(d, accs):
                    new_accs = []
                    for h in range(8):
                        flat = eidx129 + (h * 16) + d
                        kv = plsc.load_gather(kp, [flat])
                        qv = plsc.load_gather(qp, [flat])
                        new_accs.append(accs[h] + kv * qv)
                    return tuple(new_accs)
                accs = lax.fori_loop(0, 16, d_body, (zeros16,) * 8)
                for h in range(8):
                    ex = jnp.exp(accs[h] * 0.25)
                    plsc.store_scatter(
                        exb[b], [eidx, jnp.full((16,), h, jnp.int32)], ex)

            @pl.when(g >= 1)
            def _():
                pltpu.make_async_copy(vb[o], accv.at[dst_s[o]], ssem[o]).wait()
                pltpu.make_async_copy(exb[o], accd.at[dst_s[o]], ssem[o]).wait()

            @pl.when(g < NCH - 1)
            def _():
                pltpu.async_copy(v_hbm.at[src_i[o]], vb[o], vsem[o])

            pltpu.make_async_copy(v_hbm.at[src_i[b]], vb[b], vsem[b]).wait()

            def wt_body(e, _):
                ev = exb[b][e, :]
                for h in range(8):
                    x = ev[h]
                    vb[b][e, pl.ds(h * 16, 16)] = vb[b][e, pl.ds(h * 16, 16)] * x
                return _
            lax.fori_loop(0, CHUNK, wt_body, None)

            for j in range(CHUNK // 16):
                dst_s[b][pl.ds(j * 16, 16)] = dst_i[b][pl.ds(j * 16, 16)]

            pltpu.async_copy(vb[b], accv.at[dst_s[b]], ssem[b], add=True)
            pltpu.async_copy(exb[b], accd.at[dst_s[b]], ssem[b], add=True)
        return _

    lax.fori_loop(0, NCH // 2, pair_body, None)

    pltpu.make_async_copy(vb[1], accv.at[dst_s[1]], ssem[1]).wait()
    pltpu.make_async_copy(exb[1], accd.at[dst_s[1]], ssem[1]).wait()
    plsc.subcore_barrier()

    pltpu.sync_copy(accv.at[pl.ds(row0, ROWS_PER_TILE)],
                    outv_hbm.at[pl.ds(row0, ROWS_PER_TILE)])
    pltpu.sync_copy(accd.at[pl.ds(row0, ROWS_PER_TILE)],
                    outd_hbm.at[pl.ds(row0, ROWS_PER_TILE)])


def _sc_body(k0, q0, v0, k1, q1, v1, src_hbm, dst_hbm,
             numv0, numd0, numv1, numd1,
             accv, accd,
             src_i0, src_i1, dst_i0, dst_i1, dst_s0, dst_s1,
             kb0, kb1, qb0, qb1, vb0, vb1, exb0, exb1, kp, qp,
             gsem0, gsem1, vsem0, vsem1, ssem0, ssem1):
    cid = lax.axis_index("c")
    args = (accv, accd, (src_i0, src_i1), (dst_i0, dst_i1), (dst_s0, dst_s1),
            (kb0, kb1), (qb0, qb1), (vb0, vb1), (exb0, exb1), kp, qp,
            (gsem0, gsem1), (vsem0, vsem1), (ssem0, ssem1))

    @pl.when(cid == 0)
    def _():
        _sc_half(k0, q0, v0, src_hbm, dst_hbm, numv0, numd0, *args)

    @pl.when(cid == 1)
    def _():
        _sc_half(k1, q1, v1, src_hbm, dst_hbm, numv1, numd1, *args)


def _sc_edges(k0, q0, v0, k1, q1, v1, src, dst):
    mesh = plsc.VectorSubcoreMesh(core_axis_name="c", subcore_axis_name="s")
    outv = jax.ShapeDtypeStruct((N, HD), jnp.float32)
    outd = jax.ShapeDtypeStruct((N, 16), jnp.float32)
    idx_t = pltpu.VMEM((CHUNK,), jnp.int32)
    row_t = pltpu.VMEM((CHUNK, HD), jnp.float32)
    pad_t = pltpu.VMEM((CHUNK * (HD + 1),), jnp.float32)
    ex_t = pltpu.VMEM((CHUNK, 16), jnp.float32)
    sem_t = pltpu.SemaphoreType.DMA
    fn = pl.kernel(
        _sc_body,
        mesh=mesh,
        out_type=[outv, outd, outv, outd],
        compiler_params=pltpu.CompilerParams(use_tc_tiling_on_sc=False,
                                             needs_layout_passes=False),
        scratch_types=[
            pltpu.VMEM_SHARED((NP, HD), jnp.float32),
            pltpu.VMEM_SHARED((NP, 16), jnp.float32),
            idx_t, idx_t, idx_t, idx_t, idx_t, idx_t,
            row_t, row_t, row_t, row_t, row_t, row_t, ex_t, ex_t, pad_t, pad_t,
            sem_t, sem_t, sem_t, sem_t, sem_t, sem_t,
        ],
    )
    return fn(k0, q0, v0, k1, q1, v1, src, dst)


def _out_body(nv0_ref, nd0_ref, nv1_ref, nd1_ref, wa, ba, out_ref):
    f32 = jnp.float32
    row = lax.broadcasted_iota(jnp.int32, (8, HD), 0)
    col = lax.broadcasted_iota(jnp.int32, (8, HD), 1)
    expand = (col // 16 == row).astype(f32)
    r0 = 1.0 / jnp.maximum(nd0_ref[:, :8], 1e-30)
    r1 = 1.0 / jnp.maximum(nd1_ref[:, :8], 1e-30)
    att0 = nv0_ref[:, :] * jnp.dot(r0, expand, preferred_element_type=f32)
    att1 = nv1_ref[:, :] * jnp.dot(r1, expand, preferred_element_type=f32)
    out = (jnp.dot(att0, wa[:HD, :], preferred_element_type=f32)
           + jnp.dot(att1, wa[HD:, :], preferred_element_type=f32)
           + ba[:, :])
    out_ref[:, :] = out


def _output(nv0, nd0, nv1, nd1, Wa, ba):
    vspec = pl.BlockSpec((RB, HD), lambda i: (i, 0))
    dspec = pl.BlockSpec((RB, 16), lambda i: (i, 0))
    return pl.pallas_call(
        _out_body,
        grid=(NB,),
        in_specs=[vspec, dspec, vspec, dspec,
                  pl.BlockSpec((D, D), lambda i: (0, 0)),
                  pl.BlockSpec((1, D), lambda i: (0, 0))],
        out_specs=pl.BlockSpec((RB, D), lambda i: (i, 0)),
        out_shape=jax.ShapeDtypeStruct((N, D), jnp.float32),
    )(nv0, nd0, nv1, nd1, Wa, ba.reshape(1, D))


def kernel(h, edge_index, Wq, bq, Wk, bk, Wv, bv, Wm, bm, Wat, bat, Wa, ba):
    pad = jnp.full((EP - E,), N, jnp.int32)
    src = jnp.concatenate([edge_index[0], pad])
    dst = jnp.concatenate([edge_index[1], pad])
    q0, q1, k0, k1, v0, v1 = _project(h, Wq, bq, Wk, bk, Wv, bv,
                                      Wm, bm, Wat, bat)
    zrows = jnp.zeros((NP - N, HD), jnp.float32)
    q0, q1, k0, k1, v0, v1 = (jnp.concatenate([a, zrows])
                              for a in (q0, q1, k0, k1, v0, v1))
    nv0, nd0, nv1, nd1 = _sc_edges(k0, q0, v0, k1, q1, v1, src, dst)
    return _output(nv0, nd0, nv1, nd1, Wa, ba)


# speedup vs baseline: 1.6347x; 1.1209x over previous
import functools
import math

import jax
import jax.numpy as jnp
from jax import lax
from jax.experimental import pallas as pl
from jax.experimental.pallas import tpu as pltpu
from jax.experimental.pallas import tpu_sc as plsc

N = 10000
E = 160000
D = 256
HD = 128
ACC_W = 144
NB = 10
RB = N // NB

NUM_TILES = 16
NP = N + 8
CHUNK = 32
EPT = 10048
EP = EPT * NUM_TILES
NCH = EPT // CHUNK
ROWS_PER_TILE = N // NUM_TILES


def _proj_body(h_ref, wq, bq, wk, bk, wv, bv, wm, bm, wat, bat,
               q0, q1, k0, k1, v0, v1):
    hb = h_ref[:, :]
    f32 = jnp.float32
    q = jnp.dot(hb, wq[:, :], preferred_element_type=f32) + bq[:, :]
    kt = jnp.dot(hb, wk[:, :], preferred_element_type=f32) + bk[:, :]
    k = jnp.dot(kt, wat[:, :], preferred_element_type=f32) + bat[:, :]
    vt = jnp.dot(hb, wv[:, :], preferred_element_type=f32) + bv[:, :]
    v = jnp.dot(vt, wm[:, :], preferred_element_type=f32) + bm[:, :]
    q0[:, :] = q[:, :HD]
    q1[:, :] = q[:, HD:]
    k0[:, :] = k[:, :HD]
    k1[:, :] = k[:, HD:]
    v0[:, :] = v[:, :HD]
    v1[:, :] = v[:, HD:]


def _project(h, Wq, bq, Wk, bk, Wv, bv, Wm, bm, Wat, bat):
    wspec = pl.BlockSpec((D, D), lambda i: (0, 0))
    bspec = pl.BlockSpec((1, D), lambda i: (0, 0))
    hspec = pl.BlockSpec((RB, D), lambda i: (i, 0))
    ospec = pl.BlockSpec((RB, HD), lambda i: (i, 0))
    out = jax.ShapeDtypeStruct((N, HD), jnp.float32)
    return pl.pallas_call(
        _proj_body,
        grid=(NB,),
        in_specs=[hspec, wspec, bspec, wspec, bspec, wspec, bspec,
                  wspec, bspec, wspec, bspec],
        out_specs=[ospec] * 6,
        out_shape=[out] * 6,
    )(h, Wq, bq.reshape(1, D), Wk, bk.reshape(1, D), Wv, bv.reshape(1, D),
      Wm, bm.reshape(1, D), Wat, bat.reshape(1, D))


def _sc_half(k_hbm, q_hbm, v_hbm, src_hbm, dst_hbm, outv_hbm, outd_hbm,
             accv, accd, src_i, dst_i, dst_s, kb, qb, vb, exb, kp, qp,
             gsem, vsem, ssem, isem):
    sid = lax.axis_index("s")
    zeros16 = jnp.zeros((16,), jnp.float32)
    lanes = jnp.arange(16, dtype=jnp.int32)

    def zrow(e, _):
        for j in range(HD // 16):
            vb[0][e, pl.ds(j * 16, 16)] = zeros16
        exb[0][e, :] = zeros16
        return _
    lax.fori_loop(0, CHUNK, zrow, None)

    row0 = sid * ROWS_PER_TILE
    def zacc(j, _):
        pltpu.sync_copy(vb[0].at[pl.ds(0, CHUNK)],
                        accv.at[pl.ds(row0 + j * CHUNK, CHUNK)])
        pltpu.sync_copy(exb[0].at[pl.ds(0, CHUNK)],
                        accd.at[pl.ds(row0 + j * CHUNK, CHUNK)])
        return _
    lax.fori_loop(0, 19, zacc, None)
    pltpu.sync_copy(vb[0].at[pl.ds(0, 17)], accv.at[pl.ds(row0 + 608, 17)])
    pltpu.sync_copy(exb[0].at[pl.ds(0, 17)], accd.at[pl.ds(row0 + 608, 17)])

    ebase = sid * EPT

    pltpu.sync_copy(src_hbm.at[pl.ds(ebase, CHUNK)], src_i[0])
    pltpu.sync_copy(dst_hbm.at[pl.ds(ebase, CHUNK)], dst_i[0])
    pltpu.async_copy(k_hbm.at[src_i[0]], kb[0], gsem[0])
    pltpu.async_copy(q_hbm.at[dst_i[0]], qb[0], gsem[0])
    pltpu.async_copy(v_hbm.at[src_i[0]], vb[0], vsem[0])
    pltpu.async_copy(src_hbm.at[pl.ds(ebase + CHUNK, CHUNK)], src_i[1], isem[1])
    pltpu.async_copy(dst_hbm.at[pl.ds(ebase + CHUNK, CHUNK)], dst_i[1], isem[1])
    plsc.subcore_barrier()

    def pair_body(p, _):
        for b in (0, 1):
            o = 1 - b
            g = 2 * p + b

            @pl.when(g < NCH - 1)
            def _():
                nbase = ebase + (g + 1) * CHUNK
                pltpu.make_async_copy(src_hbm.at[pl.ds(nbase, CHUNK)],
                                      src_i[o], isem[o]).wait()
                pltpu.make_async_copy(dst_hbm.at[pl.ds(nbase, CHUNK)],
                                      dst_i[o], isem[o]).wait()
                pltpu.async_copy(k_hbm.at[src_i[o]], kb[o], gsem[o])
                pltpu.async_copy(q_hbm.at[dst_i[o]], qb[o], gsem[o])

            pltpu.make_async_copy(k_hbm.at[src_i[b]], kb[b], gsem[b]).wait()
            pltpu.make_async_copy(q_hbm.at[dst_i[b]], qb[b], gsem[b]).wait()
            def rs_body(e, _):
                for j in range(8):
                    kp[pl.ds(e * 129 + j * 16, 16)] = kb[b][e, pl.ds(j * 16, 16)]
                    qp[pl.ds(e * 129 + j * 16, 16)] = qb[b][e, pl.ds(j * 16, 16)]
                return _
            lax.fori_loop(0, CHUNK, rs_body, None)

            for gg in range(CHUNK // 16):
                eidx = gg * 16 + lanes

                eidx129 = eidx * 129

                def d_body(d, accs):
                    new_accs = []
                    for h in range(8):
                        flat = eidx129 + (h * 16) + d
                        kv = plsc.load_gather(kp, [flat])
                        qv = plsc.load_gather(qp, [flat])
                        new_accs.append(accs[h] + kv * qv)
                    return tuple(new_accs)
                accs = lax.fori_loop(0, 16, d_body, (zeros16,) * 8)
                for h in range(8):
                    ex = jnp.exp(accs[h] * 0.25)
                    plsc.store_scatter(
                        exb[b], [eidx, jnp.full((16,), h, jnp.int32)], ex)

            @pl.when(g >= 1)
            def _():
                pltpu.make_async_copy(vb[o], accv.at[dst_s[o]], ssem[o]).wait()
                pltpu.make_async_copy(exb[o], accd.at[dst_s[o]], ssem[o]).wait()

            @pl.when(g < NCH - 1)
            def _():
                pltpu.async_copy(v_hbm.at[src_i[o]], vb[o], vsem[o])

            pltpu.make_async_copy(v_hbm.at[src_i[b]], vb[b], vsem[b]).wait()

            def wt_body(e, _):
                ev = exb[b][e, :]
                for h in range(8):
                    x = ev[h]
                    vb[b][e, pl.ds(h * 16, 16)] = vb[b][e, pl.ds(h * 16, 16)] * x
                return _
            lax.fori_loop(0, CHUNK, wt_body, None)

            for j in range(CHUNK // 16):
                dst_s[b][pl.ds(j * 16, 16)] = dst_i[b][pl.ds(j * 16, 16)]

            pltpu.async_copy(vb[b], accv.at[dst_s[b]], ssem[b], add=True)
            pltpu.async_copy(exb[b], accd.at[dst_s[b]], ssem[b], add=True)

            @pl.when(g + 2 < NCH)
            def _():
                pbase = ebase + (g + 2) * CHUNK
                pltpu.async_copy(src_hbm.at[pl.ds(pbase, CHUNK)], src_i[b], isem[b])
                pltpu.async_copy(dst_hbm.at[pl.ds(pbase, CHUNK)], dst_i[b], isem[b])
        return _

    lax.fori_loop(0, NCH // 2, pair_body, None)

    pltpu.make_async_copy(vb[1], accv.at[dst_s[1]], ssem[1]).wait()
    pltpu.make_async_copy(exb[1], accd.at[dst_s[1]], ssem[1]).wait()
    plsc.subcore_barrier()

    pltpu.sync_copy(accv.at[pl.ds(row0, ROWS_PER_TILE)],
                    outv_hbm.at[pl.ds(row0, ROWS_PER_TILE)])
    pltpu.sync_copy(accd.at[pl.ds(row0, ROWS_PER_TILE)],
                    outd_hbm.at[pl.ds(row0, ROWS_PER_TILE)])


def _sc_body(k0, q0, v0, k1, q1, v1, src_hbm, dst_hbm,
             numv0, numd0, numv1, numd1,
             accv, accd,
             src_i0, src_i1, dst_i0, dst_i1, dst_s0, dst_s1,
             kb0, kb1, qb0, qb1, vb0, vb1, exb0, exb1, kp, qp,
             gsem0, gsem1, vsem0, vsem1, ssem0, ssem1, isem0, isem1):
    cid = lax.axis_index("c")
    args = (accv, accd, (src_i0, src_i1), (dst_i0, dst_i1), (dst_s0, dst_s1),
            (kb0, kb1), (qb0, qb1), (vb0, vb1), (exb0, exb1), kp, qp,
            (gsem0, gsem1), (vsem0, vsem1), (ssem0, ssem1), (isem0, isem1))

    @pl.when(cid == 0)
    def _():
        _sc_half(k0, q0, v0, src_hbm, dst_hbm, numv0, numd0, *args)

    @pl.when(cid == 1)
    def _():
        _sc_half(k1, q1, v1, src_hbm, dst_hbm, numv1, numd1, *args)


def _sc_edges(k0, q0, v0, k1, q1, v1, src, dst):
    mesh = plsc.VectorSubcoreMesh(core_axis_name="c", subcore_axis_name="s")
    outv = jax.ShapeDtypeStruct((N, HD), jnp.float32)
    outd = jax.ShapeDtypeStruct((N, 16), jnp.float32)
    idx_t = pltpu.VMEM((CHUNK,), jnp.int32)
    row_t = pltpu.VMEM((CHUNK, HD), jnp.float32)
    pad_t = pltpu.VMEM((CHUNK * (HD + 1),), jnp.float32)
    ex_t = pltpu.VMEM((CHUNK, 16), jnp.float32)
    sem_t = pltpu.SemaphoreType.DMA
    fn = pl.kernel(
        _sc_body,
        mesh=mesh,
        out_type=[outv, outd, outv, outd],
        compiler_params=pltpu.CompilerParams(use_tc_tiling_on_sc=False,
                                             needs_layout_passes=False),
        scratch_types=[
            pltpu.VMEM_SHARED((NP, HD), jnp.float32),
            pltpu.VMEM_SHARED((NP, 16), jnp.float32),
            idx_t, idx_t, idx_t, idx_t, idx_t, idx_t,
            row_t, row_t, row_t, row_t, row_t, row_t, ex_t, ex_t, pad_t, pad_t,
            sem_t, sem_t, sem_t, sem_t, sem_t, sem_t, sem_t, sem_t,
        ],
    )
    return fn(k0, q0, v0, k1, q1, v1, src, dst)


def _out_body(nv0_ref, nd0_ref, nv1_ref, nd1_ref, wa, ba, out_ref):
    f32 = jnp.float32
    row = lax.broadcasted_iota(jnp.int32, (8, HD), 0)
    col = lax.broadcasted_iota(jnp.int32, (8, HD), 1)
    expand = (col // 16 == row).astype(f32)
    r0 = 1.0 / jnp.maximum(nd0_ref[:, :8], 1e-30)
    r1 = 1.0 / jnp.maximum(nd1_ref[:, :8], 1e-30)
    att0 = nv0_ref[:, :] * jnp.dot(r0, expand, preferred_element_type=f32)
    att1 = nv1_ref[:, :] * jnp.dot(r1, expand, preferred_element_type=f32)
    out = (jnp.dot(att0, wa[:HD, :], preferred_element_type=f32)
           + jnp.dot(att1, wa[HD:, :], preferred_element_type=f32)
           + ba[:, :])
    out_ref[:, :] = out


def _output(nv0, nd0, nv1, nd1, Wa, ba):
    vspec = pl.BlockSpec((RB, HD), lambda i: (i, 0))
    dspec = pl.BlockSpec((RB, 16), lambda i: (i, 0))
    return pl.pallas_call(
        _out_body,
        grid=(NB,),
        in_specs=[vspec, dspec, vspec, dspec,
                  pl.BlockSpec((D, D), lambda i: (0, 0)),
                  pl.BlockSpec((1, D), lambda i: (0, 0))],
        out_specs=pl.BlockSpec((RB, D), lambda i: (i, 0)),
        out_shape=jax.ShapeDtypeStruct((N, D), jnp.float32),
    )(nv0, nd0, nv1, nd1, Wa, ba.reshape(1, D))


def kernel(h, edge_index, Wq, bq, Wk, bk, Wv, bv, Wm, bm, Wat, bat, Wa, ba):
    pad = jnp.full((EP - E,), N, jnp.int32)
    src = jnp.concatenate([edge_index[0], pad])
    dst = jnp.concatenate([edge_index[1], pad])
    q0, q1, k0, k1, v0, v1 = _project(h, Wq, bq, Wk, bk, Wv, bv,
                                      Wm, bm, Wat, bat)
    zrows = jnp.zeros((NP - N, HD), jnp.float32)
    q0, q1, k0, k1, v0, v1 = (jnp.concatenate([a, zrows])
                              for a in (q0, q1, k0, k1, v0, v1))
    nv0, nd0, nv1, nd1 = _sc_edges(k0, q0, v0, k1, q1, v1, src, dst)
    return _output(nv0, nd0, nv1, nd1, Wa, ba)
